# baseline (speedup 1.0000x reference)
import jax
import jax.numpy as jnp
from jax.experimental import pallas as pl
from jax.experimental.pallas import tpu as pltpu


W_HALVES = 2


def _mm_kernel(idx_ref, x_ref, *rest):
    del idx_ref
    nw = len(rest) - 3
    w_refs = rest[:nw]
    b_refs = rest[nw:nw + 2]
    o_ref = rest[-1]
    s_slots = len(b_refs)
    nh = nw // s_slots
    ob = o_ref.shape[-1] // nh
    i = pl.program_id(0)
    xa = x_ref[i]
    for s in range(s_slots):
        for h in range(nh):
            acc = jax.lax.dot_general(
                xa, w_refs[s * nh + h][0, 0],
                dimension_numbers=(((1,), (1,)), ((), ())),
                preferred_element_type=jnp.float32,
            )
            o_ref[0, s, :, h * ob:(h + 1) * ob] = (
                acc + b_refs[s][0, 0, h * ob:(h + 1) * ob])


def kernel(x, indices, weight, bias):
    G, S = indices.shape
    E, OUT_D, IN_D = weight.shape
    T = x.shape[-2]
    OB = OUT_D // W_HALVES

    xr = x.reshape(G, T, IN_D)
    ws = weight.reshape(E, W_HALVES, OB, IN_D)
    br = bias.reshape(E, 1, OUT_D)

    def _wmap(s, h):
        return lambda i, ind: (ind[i, s], h, 0, 0)

    def _bmap(s):
        return lambda i, ind: (ind[i, s], 0, 0)

    grid_spec = pltpu.PrefetchScalarGridSpec(
        num_scalar_prefetch=1,
        grid=(G,),
        in_specs=[
            pl.BlockSpec((G, T, IN_D), lambda i, ind: (0, 0, 0)),
        ] + [
            pl.BlockSpec((1, 1, OB, IN_D), _wmap(s, h))
            for s in range(S) for h in range(W_HALVES)
        ] + [
            pl.BlockSpec((1, 1, OUT_D), _bmap(s)) for s in range(S)
        ],
        out_specs=pl.BlockSpec((1, S, T, OUT_D),
                               lambda i, ind: (i, 0, 0, 0)),
    )

    out = pl.pallas_call(
        _mm_kernel,
        grid_spec=grid_spec,
        out_shape=jax.ShapeDtypeStruct((G, S, T, OUT_D), jnp.float32),
    )(indices, xr, *([ws] * (S * W_HALVES)), *([br] * S))

    return out.reshape(1, G, S, T, OUT_D)


# speedup vs baseline: 1.1769x; 1.1769x over previous
import jax
import jax.numpy as jnp
from jax.experimental import pallas as pl
from jax.experimental.pallas import tpu as pltpu

_XCHUNKS = 4


def _mm_kernel(idx_ref, fu_ref, eord_ref, mask_ref,
               x_hbm, w_hbm, b_ref, o_ref,
               xscr, wscr, xsem, wsem):
    G, T, IN_D = xscr.shape
    E = wscr.shape[0]
    S = idx_ref.shape[1]
    rows = G // _XCHUNKS
    i = pl.program_id(0)

    @pl.when(i == 0)
    def _issue():
        for c in range(_XCHUNKS):
            pltpu.make_async_copy(
                x_hbm.at[pl.ds(c * rows, rows)],
                xscr.at[pl.ds(c * rows, rows)],
                xsem.at[c],
            ).start()
        for k in range(E):
            e = eord_ref[k]

            @pl.when(mask_ref[k] == 1)
            def _start_w():
                pltpu.make_async_copy(
                    w_hbm.at[e], wscr.at[e], wsem.at[e]).start()

    @pl.when(i % rows == 0)
    def _wait_x():
        c = i // rows
        pltpu.make_async_copy(
            x_hbm.at[pl.ds(c * rows, rows)],
            xscr.at[pl.ds(c * rows, rows)],
            xsem.at[c],
        ).wait()

    for s in range(S):
        e_s = idx_ref[i, s]

        @pl.when(fu_ref[i, s] == 1)
        def _wait_w():
            pltpu.make_async_copy(
                w_hbm.at[e_s], wscr.at[e_s], wsem.at[e_s]).wait()

        acc = jax.lax.dot_general(
            xscr[i], wscr[e_s],
            dimension_numbers=(((1,), (1,)), ((), ())),
            preferred_element_type=jnp.float32,
        )
        o_ref[0, s] = acc + b_ref[e_s]


def kernel(x, indices, weight, bias):
    G, S = indices.shape
    E, OUT_D, IN_D = weight.shape
    T = x.shape[-2]
    P = G * S

    xr = x.reshape(G, T, IN_D)

    flat = indices.reshape(P)
    eq = flat[:, None] == flat[None, :]
    first = jnp.argmax(eq, axis=1)
    fu = (first == jnp.arange(P)).astype(jnp.int32).reshape(G, S)
    onehot = flat[None, :] == jnp.arange(E)[:, None]
    firstpos = jnp.where(onehot, jnp.arange(P)[None, :], P).min(axis=1)
    eord = jnp.argsort(firstpos).astype(jnp.int32)
    mask = (jnp.sort(firstpos) < P).astype(jnp.int32)

    grid_spec = pltpu.PrefetchScalarGridSpec(
        num_scalar_prefetch=4,
        grid=(G,),
        in_specs=[
            pl.BlockSpec(memory_space=pl.ANY),
            pl.BlockSpec(memory_space=pl.ANY),
            pl.BlockSpec((E, OUT_D), lambda i, *_: (0, 0)),
        ],
        out_specs=pl.BlockSpec((1, S, T, OUT_D),
                               lambda i, *_: (i, 0, 0, 0)),
        scratch_shapes=[
            pltpu.VMEM((G, T, IN_D), jnp.float32),
            pltpu.VMEM((E, OUT_D, IN_D), jnp.float32),
            pltpu.SemaphoreType.DMA((_XCHUNKS,)),
            pltpu.SemaphoreType.DMA((E,)),
        ],
    )

    out = pl.pallas_call(
        _mm_kernel,
        grid_spec=grid_spec,
        out_shape=jax.ShapeDtypeStruct((G, S, T, OUT_D), jnp.float32),
    )(indices, fu, eord, mask, xr, weight, bias)

    return out.reshape(1, G, S, T, OUT_D)
